# baseline (speedup 1.0000x reference)
import functools

import jax
import jax.numpy as jnp
from jax import lax
from jax.experimental import pallas as pl
from jax.experimental.pallas import tpu as pltpu
from jax.experimental.pallas import tpu_sc as plsc

_LANES = 16


@functools.lru_cache(maxsize=None)
def _build(num_rows, vocab, hidden, max_pos):
    info = plsc.get_sparse_core_info()
    num_workers = info.num_cores * info.num_subcores
    assert num_rows % num_workers == 0
    rows_per_worker = num_rows // num_workers
    chunk = 32
    assert rows_per_worker % chunk == 0
    num_chunks = rows_per_worker // chunk
    assert hidden % _LANES == 0
    vecs_per_row = hidden // _LANES

    mesh = plsc.VectorSubcoreMesh(core_axis_name="c", subcore_axis_name="s")

    def body(ids_hbm, table_hbm, pos_hbm, out_hbm, idx_v, pos_v, buf, sem):
        wid = lax.axis_index("s") * info.num_cores + lax.axis_index("c")
        base = wid * rows_per_worker
        pltpu.sync_copy(ids_hbm.at[pl.ds(base, rows_per_worker)], idx_v)
        pltpu.sync_copy(pos_hbm, pos_v)

        @pl.loop(0, num_chunks)
        def chunk_loop(c):
            idx_slice = idx_v.at[pl.ds(c * chunk, chunk)]
            pltpu.async_copy(table_hbm.at[idx_slice], buf, sem).wait()

            @pl.loop(0, chunk)
            def row_loop(i):
                p = lax.rem(c * chunk + i, max_pos)
                for j in range(vecs_per_row):
                    sl = pl.ds(j * _LANES, _LANES)
                    plsc.addupdate(buf.at[i, sl], pos_v[p, sl])

            pltpu.sync_copy(buf, out_hbm.at[pl.ds(base + c * chunk, chunk)])

    return pl.kernel(
        body,
        out_type=jax.ShapeDtypeStruct((num_rows, hidden), jnp.float32),
        mesh=mesh,
        scratch_types=[
            pltpu.VMEM((rows_per_worker,), jnp.int32),
            pltpu.VMEM((max_pos, hidden), jnp.float32),
            pltpu.VMEM((chunk, hidden), jnp.float32),
            pltpu.SemaphoreType.DMA,
        ],
    )


def kernel(input_ids, token_table, pos_table):
    batch, seq_len = input_ids.shape
    vocab, hidden = token_table.shape
    max_pos = pos_table.shape[0]
    assert seq_len == max_pos
    ids = input_ids.reshape(-1).astype(jnp.int32)
    fn = _build(batch * seq_len, vocab, hidden, max_pos)
    out = fn(ids, token_table, pos_table)
    return out.reshape(batch, seq_len, hidden)


# speedup vs baseline: 1.1663x; 1.1663x over previous
import functools

import jax
import jax.numpy as jnp
from jax import lax
from jax.experimental import pallas as pl
from jax.experimental.pallas import tpu as pltpu
from jax.experimental.pallas import tpu_sc as plsc

_LANES = 16


@functools.lru_cache(maxsize=None)
def _build(num_rows, vocab, hidden, max_pos):
    info = plsc.get_sparse_core_info()
    num_workers = info.num_cores * info.num_subcores
    assert num_rows % num_workers == 0
    rows_per_worker = num_rows // num_workers
    chunk = 32
    assert rows_per_worker % chunk == 0
    num_chunks = rows_per_worker // chunk
    assert hidden % _LANES == 0
    vecs_per_row = hidden // _LANES

    mesh = plsc.VectorSubcoreMesh(core_axis_name="c", subcore_axis_name="s")

    def body(ids_hbm, table_hbm, pos_hbm, out_hbm, idx_v, pos_v, buf, sem):
        wid = lax.axis_index("s") * info.num_cores + lax.axis_index("c")
        base = wid * rows_per_worker
        pltpu.sync_copy(ids_hbm.at[pl.ds(base, rows_per_worker)], idx_v)
        pltpu.sync_copy(pos_hbm, pos_v)

        def gather(c, slot):
            idx_slice = idx_v.at[pl.ds(c * chunk, chunk)]
            return pltpu.make_async_copy(
                table_hbm.at[idx_slice], buf.at[slot], sem.at[slot]
            )

        gather(0, 0).start()

        @pl.loop(0, num_chunks)
        def chunk_loop(c):
            slot = lax.rem(c, 2)

            @pl.when(c + 1 < num_chunks)
            def _():
                gather(c + 1, 1 - slot).start()

            gather(c, slot).wait()

            @pl.loop(0, chunk)
            def row_loop(i):
                p = lax.rem(c * chunk + i, max_pos)
                for j in range(vecs_per_row):
                    sl = pl.ds(j * _LANES, _LANES)
                    plsc.addupdate(buf.at[slot, i, sl], pos_v[p, sl])

            pltpu.sync_copy(buf.at[slot], out_hbm.at[pl.ds(base + c * chunk, chunk)])

    return pl.kernel(
        body,
        out_type=jax.ShapeDtypeStruct((num_rows, hidden), jnp.float32),
        mesh=mesh,
        scratch_types=[
            pltpu.VMEM((rows_per_worker,), jnp.int32),
            pltpu.VMEM((max_pos, hidden), jnp.float32),
            pltpu.VMEM((2, chunk, hidden), jnp.float32),
            pltpu.SemaphoreType.DMA((2,)),
        ],
    )


def kernel(input_ids, token_table, pos_table):
    batch, seq_len = input_ids.shape
    vocab, hidden = token_table.shape
    max_pos = pos_table.shape[0]
    assert seq_len == max_pos
    ids = input_ids.reshape(-1).astype(jnp.int32)
    fn = _build(batch * seq_len, vocab, hidden, max_pos)
    out = fn(ids, token_table, pos_table)
    return out.reshape(batch, seq_len, hidden)


# speedup vs baseline: 1.6819x; 1.4421x over previous
import functools

import jax
import jax.numpy as jnp
from jax import lax
from jax.experimental import pallas as pl
from jax.experimental.pallas import tpu as pltpu
from jax.experimental.pallas import tpu_sc as plsc

_LANES = 16


@functools.lru_cache(maxsize=None)
def _build(num_rows, vocab, hidden, max_pos):
    info = plsc.get_sparse_core_info()
    num_workers = info.num_cores * info.num_subcores
    assert num_rows % num_workers == 0
    rows_per_worker = num_rows // num_workers
    chunk = 32
    assert rows_per_worker % chunk == 0
    num_chunks = rows_per_worker // chunk
    assert hidden % _LANES == 0
    vecs_per_row = hidden // _LANES

    mesh = plsc.VectorSubcoreMesh(core_axis_name="c", subcore_axis_name="s")

    def body(ids_hbm, table_hbm, pos_hbm, out_hbm, idx_v, pos_v, buf, gsem, ssem):
        wid = lax.axis_index("s") * info.num_cores + lax.axis_index("c")
        base = wid * rows_per_worker
        pltpu.sync_copy(ids_hbm.at[pl.ds(base, rows_per_worker)], idx_v)
        pltpu.sync_copy(pos_hbm, pos_v)

        def gather(c, slot):
            idx_slice = idx_v.at[pl.ds(c * chunk, chunk)]
            return pltpu.make_async_copy(
                table_hbm.at[idx_slice], buf.at[slot], gsem.at[slot]
            )

        def scatter(c, slot):
            return pltpu.make_async_copy(
                buf.at[slot],
                out_hbm.at[pl.ds(base + c * chunk, chunk)],
                ssem.at[slot],
            )

        gather(0, 0).start()

        @pl.loop(0, num_chunks)
        def chunk_loop(c):
            slot = lax.rem(c, 2)
            other = 1 - slot

            @pl.when(c >= 1)
            def _():
                scatter(c - 1, other).wait()

            @pl.when(c + 1 < num_chunks)
            def _():
                gather(c + 1, other).start()

            gather(c, slot).wait()

            @plsc.parallel_loop(0, chunk, unroll=2)
            def row_loop(i):
                p = lax.rem(c * chunk + i, max_pos)
                for j in range(vecs_per_row):
                    sl = pl.ds(j * _LANES, _LANES)
                    plsc.addupdate(buf.at[slot, i, sl], pos_v[p, sl])

            scatter(c, slot).start()

        scatter(num_chunks - 1, lax.rem(num_chunks - 1, 2)).wait()

    return pl.kernel(
        body,
        out_type=jax.ShapeDtypeStruct((num_rows, hidden), jnp.float32),
        mesh=mesh,
        scratch_types=[
            pltpu.VMEM((rows_per_worker,), jnp.int32),
            pltpu.VMEM((max_pos, hidden), jnp.float32),
            pltpu.VMEM((2, chunk, hidden), jnp.float32),
            pltpu.SemaphoreType.DMA((2,)),
            pltpu.SemaphoreType.DMA((2,)),
        ],
    )


def kernel(input_ids, token_table, pos_table):
    batch, seq_len = input_ids.shape
    vocab, hidden = token_table.shape
    max_pos = pos_table.shape[0]
    assert seq_len == max_pos
    ids = input_ids.reshape(-1).astype(jnp.int32)
    fn = _build(batch * seq_len, vocab, hidden, max_pos)
    out = fn(ids, token_table, pos_table)
    return out.reshape(batch, seq_len, hidden)
